# baseline (speedup 1.0000x reference)
import functools

import jax
import jax.numpy as jnp
from jax import lax
from jax.experimental import pallas as pl
from jax.experimental.pallas import tpu as pltpu
from jax.experimental.pallas import tpu_sc as plsc

NC = 2
NS = 16
NW = NC * NS
D = 32
DP = 128
CB = 128


@functools.lru_cache(maxsize=None)
def _gather_kernel(B, H):
    b_per_w = B // NW
    n_chunks = b_per_w // CB
    n_tiles = n_chunks * H
    mesh = plsc.VectorSubcoreMesh(
        core_axis_name="c", subcore_axis_name="s",
        num_cores=NC, num_subcores=NS)

    @functools.partial(
        pl.kernel,
        out_type=jax.ShapeDtypeStruct((H, D, B), jnp.float32),
        mesh=mesh,
        scratch_types=[
            pltpu.VMEM((b_per_w * H,), jnp.int32),
            pltpu.VMEM((CB, DP), jnp.float32),
            pltpu.VMEM((CB, DP), jnp.float32),
            pltpu.VMEM((D, CB), jnp.float32),
            pltpu.SemaphoreType.DMA,
            pltpu.SemaphoreType.DMA,
            pltpu.SemaphoreType.DMA,
        ],
        compiler_params=pltpu.CompilerParams(
            use_tc_tiling_on_sc=True, needs_layout_passes=False),
    )
    def k(idx_hbm, table_hbm, out_hbm, idx_v, gb0, gb1, ob, gs0, gs1, isem):
        wid = lax.axis_index("s") * NC + lax.axis_index("c")
        b0w = wid * b_per_w

        icopies = []
        for j in range(n_tiles):
            t = j % H
            cn = j // H
            icopies.append(pltpu.async_copy(
                idx_hbm.at[t, pl.ds(b0w + cn * CB, CB)],
                idx_v.at[pl.ds(j * CB, CB)], isem))
        for c in icopies:
            c.wait()

        rows_h = [lax.iota(jnp.int32, 16) + 16 * h for h in range(8)]
        cols_d = [jnp.full((16,), d, jnp.int32) for d in range(D)]

        def gather_start(j, gb, gs):
            jc = jnp.minimum(j, n_tiles - 1)
            return pltpu.async_copy(
                table_hbm.at[idx_v.at[pl.ds(jc * CB, CB)]], gb, gs)

        def transpose_store(j, gb):
            for d in range(D):
                for h in range(8):
                    v = plsc.load_gather(gb, [rows_h[h], cols_d[d]])
                    ob[d, pl.ds(16 * h, 16)] = v
            t = j % H
            cn = j // H
            pltpu.sync_copy(ob, out_hbm.at[t, :, pl.ds(b0w + cn * CB, CB)])

        gather_start(0, gb0, gs0)

        def body(s, carry):
            ja = 2 * s
            gather_start(ja + 1, gb1, gs1)
            pltpu.make_async_copy(
                table_hbm.at[pl.ds(0, CB)], gb0, gs0).wait()
            transpose_store(ja, gb0)
            gather_start(ja + 2, gb0, gs0)
            pltpu.make_async_copy(
                table_hbm.at[pl.ds(0, CB)], gb1, gs1).wait()
            transpose_store(ja + 1, gb1)
            return carry

        lax.fori_loop(0, n_tiles // 2, body, 0)
        pltpu.make_async_copy(table_hbm.at[pl.ds(0, CB)], gb0, gs0).wait()

    return k


def kernel(inputs, embedding_weight):
    B, H = inputs.shape
    idx_t = inputs.T.astype(jnp.int32)
    table_p = jnp.pad(embedding_weight, ((0, 0), (0, DP - D)))
    out_t = _gather_kernel(B, H)(idx_t, table_p)
    return lax.transpose(out_t, (2, 0, 1))
